# baseline (speedup 1.0000x reference)
import jax
import jax.numpy as jnp
from jax import lax
from jax.experimental import pallas as pl
from jax.experimental.pallas import tpu as pltpu
from jax.experimental.pallas import tpu_sc as plsc

E = 320000
D = 128
T = 640000

NW = 32
NT = T // NW
C = 160
NCH = NT // C

_SC_PARAMS = pltpu.CompilerParams(needs_layout_passes=False)


def _matmul3_body(f_ref, wt_ref, b_ref, o0, o1, o2):
    f = f_ref[...]
    o0[...] = jnp.dot(f, wt_ref[0:D, :], preferred_element_type=jnp.float32)
    o1[...] = jnp.dot(f, wt_ref[D:2 * D, :], preferred_element_type=jnp.float32)
    o2[...] = (jnp.dot(f, wt_ref[2 * D:3 * D, :],
                       preferred_element_type=jnp.float32) + b_ref[...])


def _matmul3(f_edge, Wt, b2):
    BE = 4000
    out = jax.ShapeDtypeStruct((E, D), jnp.float32)
    return pl.pallas_call(
        _matmul3_body,
        grid=(E // BE,),
        in_specs=[
            pl.BlockSpec((BE, D), lambda i: (i, 0)),
            pl.BlockSpec((3 * D, D), lambda i: (0, 0)),
            pl.BlockSpec((1, D), lambda i: (0, 0)),
        ],
        out_specs=[
            pl.BlockSpec((BE, D), lambda i: (i, 0)),
            pl.BlockSpec((BE, D), lambda i: (i, 0)),
            pl.BlockSpec((BE, D), lambda i: (i, 0)),
        ],
        out_shape=[out, out, out],
    )(f_edge, Wt, b2)


def _gelu_body(s_ref, o_ref):
    x = s_ref[...]
    o_ref[...] = 0.5 * x * (1.0 + lax.erf(x * 0.7071067811865476))


def _gelu_dense(s):
    BT = 4000
    return pl.pallas_call(
        _gelu_body,
        grid=(T // BT,),
        in_specs=[pl.BlockSpec((BT, D), lambda i: (i, 0))],
        out_specs=pl.BlockSpec((BT, D), lambda i: (i, 0)),
        out_shape=jax.ShapeDtypeStruct((T, D), jnp.float32),
    )(s)


def _gather_gelu_body(g0_hbm, g1_hbm, g2_hbm, i0_hbm, i1_hbm, i2_hbm,
                      h_hbm,
                      i0a, i1a, i2a, r0a, r1a, r2a, sem_a,
                      i0b, i1b, i2b, r0b, r1b, r2b, sem_b):
    wid = lax.axis_index("s") * 2 + lax.axis_index("c")
    base = wid * NT
    bufs = ((i0a, i1a, i2a, r0a, r1a, r2a, sem_a),
            (i0b, i1b, i2b, r0b, r1b, r2b, sem_b))

    def fire(k, b):
        i0v, i1v, i2v, r0, r1, r2, sem = bufs[b]
        off = base + k * C
        pltpu.sync_copy(i0_hbm.at[pl.ds(off, C)], i0v)
        pltpu.sync_copy(i1_hbm.at[pl.ds(off, C)], i1v)
        pltpu.sync_copy(i2_hbm.at[pl.ds(off, C)], i2v)
        pltpu.async_copy(g0_hbm.at[i0v], r0, sem)
        pltpu.async_copy(g1_hbm.at[i1v], r1, sem)
        pltpu.async_copy(g2_hbm.at[i2v], r2, sem)

    def drain(b):
        _, _, _, r0, r1, r2, sem = bufs[b]
        pltpu.make_async_copy(g0_hbm.at[pl.ds(0, C)], r0, sem).wait()
        pltpu.make_async_copy(g1_hbm.at[pl.ds(0, C)], r1, sem).wait()
        pltpu.make_async_copy(g2_hbm.at[pl.ds(0, C)], r2, sem).wait()

    def compute_store(k, b):
        _, _, _, r0, r1, r2, _ = bufs[b]

        def row(i, carry2):
            for j in range(D // 16):
                sl = pl.ds(j * 16, 16)
                r0[i, sl] = r0[i, sl] + r1[i, sl] + r2[i, sl]
            return carry2

        lax.fori_loop(0, C, row, 0, unroll=4)
        pltpu.sync_copy(r0, h_hbm.at[pl.ds(base + k * C, C)])

    fire(0, 0)

    def pair(g, carry):
        k0 = g * 2
        fire(k0 + 1, 1)
        drain(0)
        compute_store(k0, 0)
        fire(k0 + 2, 0)
        drain(1)
        compute_store(k0 + 1, 1)
        return carry

    lax.fori_loop(0, (NCH - 1) // 2, pair, 0)
    drain(0)
    compute_store(NCH - 1, 0)


def _gather_gelu(g0, g1, g2, i0, i1, i2):
    ibuf = pltpu.VMEM((C,), jnp.int32)
    rbuf = pltpu.VMEM((C, D), jnp.float32)
    kern = pl.kernel(
        _gather_gelu_body,
        out_type=jax.ShapeDtypeStruct((T, D), jnp.float32),
        mesh=plsc.VectorSubcoreMesh(core_axis_name="c", subcore_axis_name="s"),
        scratch_types=[
            ibuf, ibuf, ibuf, rbuf, rbuf, rbuf, pltpu.SemaphoreType.DMA,
            ibuf, ibuf, ibuf, rbuf, rbuf, rbuf, pltpu.SemaphoreType.DMA,
        ],
    )
    return kern(g0, g1, g2, i0, i1, i2)


CH = 10880
NP = 15
EP = NP * 2 * CH
CB = 128
C2 = 1600
SCAN = T // 16
S = 4160
FB = 3840
NF = FB // CB
MY = CH // 16
ZB = 136


def _scatter_body(h_hbm, i2_hbm, sums_hbm,
                  i2_v, cidx_st, cdst_st,
                  cdst_a, rows_a, sem_a, cdst_b, rows_b, sem_b, acc):
    c = lax.axis_index("c")
    s = lax.axis_index("s")
    scan0 = s * SCAN
    my0 = s * MY
    bufs = ((cdst_a, rows_a, sem_a), (cdst_b, rows_b, sem_b))

    def zero_rows(i, carry):
        for j in range(D // 16):
            rows_a[i, pl.ds(j * 16, 16)] = jnp.zeros((16,), jnp.float32)
        return carry

    def zero_my_slice():
        lax.fori_loop(0, ZB, zero_rows, 0)
        for k in range(MY // ZB):
            pltpu.sync_copy(rows_a.at[pl.ds(0, ZB)],
                            acc.at[pl.ds(my0 + k * ZB, ZB)])

    def fire(i, b):
        _, rows, sem = bufs[b]
        pltpu.async_copy(h_hbm.at[cidx_st.at[pl.ds(i * CB, CB)]], rows, sem)

    def drain(b):
        _, rows, sem = bufs[b]
        pltpu.make_async_copy(h_hbm.at[pl.ds(0, CB)], rows, sem).wait()

    def consume(i, b):
        cdst, rows, _ = bufs[b]
        for k in range(CB // 16):
            cdst[pl.ds(k * 16, 16)] = cdst_st[pl.ds(i * CB + k * 16, 16)]
        pltpu.sync_copy(rows, acc.at[cdst], add=True)

    def flush_all():
        fire(0, 0)
        for i in range(NF):
            if i + 1 < NF:
                fire(i + 1, (i + 1) % 2)
            drain(i % 2)
            consume(i, i % 2)

    def do_batch(bi, carry2):
        for k in range(CB // 16):
            cdst_a[pl.ds(k * 16, 16)] = cdst_st[pl.ds(bi * CB + k * 16, 16)]
        pltpu.async_copy(h_hbm.at[cidx_st.at[pl.ds(bi * CB, CB)]],
                         rows_a, sem_a).wait()
        pltpu.sync_copy(rows_a, acc.at[cdst_a], add=True)
        return carry2

    zero_my_slice()
    plsc.subcore_barrier()

    def one_pass(p, carry):
        base = (p * 2 + c) * CH

        def chunkfn(k, cnt):
            off = scan0 + k * C2
            pltpu.sync_copy(i2_hbm.at[pl.ds(off, C2)], i2_v)

            def vec(j, cnt2):
                v = i2_v[pl.ds(j * 16, 16)]
                t = off + j * 16 + lax.iota(jnp.int32, 16)
                m = (v >= base) & (v < base + CH)
                plsc.store_compressed(cidx_st.at[pl.ds(cnt2, 16)], t, mask=m)
                plsc.store_compressed(cdst_st.at[pl.ds(cnt2, 16)], v - base,
                                      mask=m)
                cnt2 = cnt2 + jnp.sum(m.astype(jnp.int32))

                @pl.when(cnt2 >= FB)
                def flush():
                    flush_all()
                    cidx_st[pl.ds(0, 16)] = cidx_st[pl.ds(FB, 16)]
                    cdst_st[pl.ds(0, 16)] = cdst_st[pl.ds(FB, 16)]

                return jnp.where(cnt2 >= FB, cnt2 - FB, cnt2)

            return lax.fori_loop(0, C2 // 16, vec, cnt)

        cnt = lax.fori_loop(0, SCAN // C2, chunkfn, 0)

        spread = s * 16 + lax.iota(jnp.int32, 16)
        for kk in range(CB // 16):
            cidx_st[pl.ds(cnt + kk * 16, 16)] = spread
            cdst_st[pl.ds(cnt + kk * 16, 16)] = jnp.full((16,), CH, jnp.int32)
        nb = (cnt + CB - 1) >> 7
        lax.fori_loop(0, nb, do_batch, 0)
        plsc.subcore_barrier()

        pltpu.sync_copy(acc.at[pl.ds(my0, MY)],
                        sums_hbm.at[pl.ds(base + my0, MY)])
        zero_my_slice()
        plsc.subcore_barrier()
        return carry

    lax.fori_loop(0, NP, one_pass, 0)


def _scatter(h, i2):
    kern = pl.kernel(
        _scatter_body,
        out_type=jax.ShapeDtypeStruct((EP, D), jnp.float32),
        mesh=plsc.VectorSubcoreMesh(core_axis_name="c", subcore_axis_name="s"),
        scratch_types=[
            pltpu.VMEM((C2,), jnp.int32),
            pltpu.VMEM((S,), jnp.int32),
            pltpu.VMEM((S,), jnp.int32),
            pltpu.VMEM((CB,), jnp.int32),
            pltpu.VMEM((CB, D), jnp.float32),
            pltpu.SemaphoreType.DMA,
            pltpu.VMEM((CB,), jnp.int32),
            pltpu.VMEM((CB, D), jnp.float32),
            pltpu.SemaphoreType.DMA,
            pltpu.VMEM_SHARED((CH + 1, D), jnp.float32),
        ],
        compiler_params=_SC_PARAMS,
    )
    return kern(h, i2)


def _finalize_body(s_ref, g_ref, bt_ref, o_ref):
    x = s_ref[...].astype(jnp.float32)
    mu = jnp.mean(x, axis=-1, keepdims=True)
    var = jnp.mean((x - mu) ** 2, axis=-1, keepdims=True)
    o_ref[...] = (x - mu) * lax.rsqrt(var + 1e-5) * g_ref[...] + bt_ref[...]


def _finalize(sums, gamma, beta):
    BE2 = 4000
    return pl.pallas_call(
        _finalize_body,
        grid=(E // BE2,),
        in_specs=[
            pl.BlockSpec((BE2, D), lambda i: (i, 0)),
            pl.BlockSpec((1, D), lambda i: (0, 0)),
            pl.BlockSpec((1, D), lambda i: (0, 0)),
        ],
        out_specs=pl.BlockSpec((BE2, D), lambda i: (i, 0)),
        out_shape=jax.ShapeDtypeStruct((E, D), jnp.float32),
    )(sums, gamma.reshape(1, D), beta.reshape(1, D))


def kernel(f_edge, triangle, W, b, gamma, beta):
    tri = triangle.astype(jnp.int32)
    i0 = tri[:, 0]
    i1 = tri[:, 1]
    i2 = tri[:, 2]
    Wt = W.T
    b2 = b.reshape(1, D)

    g0, g1, g2 = _matmul3(f_edge, Wt, b2)
    s = _gather_gelu(g0, g1, g2, i0, i1, i2)
    h = _gelu_dense(s)
    sums = _scatter(h, i2)
    return _finalize(sums, gamma, beta)


# speedup vs baseline: 1.0533x; 1.0533x over previous
import jax
import jax.numpy as jnp
from jax import lax
from jax.experimental import pallas as pl
from jax.experimental.pallas import tpu as pltpu
from jax.experimental.pallas import tpu_sc as plsc

E = 320000
D = 128
T = 640000

NW = 32
NT = T // NW
C = 160
NCH = NT // C

_SC_PARAMS = pltpu.CompilerParams(needs_layout_passes=False)


def _matmul3_body(f_ref, wt_ref, b_ref, o0, o1, o2):
    f = f_ref[...]
    o0[...] = jnp.dot(f, wt_ref[0:D, :], preferred_element_type=jnp.float32)
    o1[...] = jnp.dot(f, wt_ref[D:2 * D, :], preferred_element_type=jnp.float32)
    o2[...] = (jnp.dot(f, wt_ref[2 * D:3 * D, :],
                       preferred_element_type=jnp.float32) + b_ref[...])


def _matmul3(f_edge, Wt, b2):
    BE = 4000
    out = jax.ShapeDtypeStruct((E, D), jnp.float32)
    return pl.pallas_call(
        _matmul3_body,
        grid=(E // BE,),
        in_specs=[
            pl.BlockSpec((BE, D), lambda i: (i, 0)),
            pl.BlockSpec((3 * D, D), lambda i: (0, 0)),
            pl.BlockSpec((1, D), lambda i: (0, 0)),
        ],
        out_specs=[
            pl.BlockSpec((BE, D), lambda i: (i, 0)),
            pl.BlockSpec((BE, D), lambda i: (i, 0)),
            pl.BlockSpec((BE, D), lambda i: (i, 0)),
        ],
        out_shape=[out, out, out],
    )(f_edge, Wt, b2)


def _gelu_body(s_ref, o_ref):
    x = s_ref[...]
    o_ref[...] = 0.5 * x * (1.0 + lax.erf(x * 0.7071067811865476))


def _gelu_dense(s):
    BT = 4000
    return pl.pallas_call(
        _gelu_body,
        grid=(T // BT,),
        in_specs=[pl.BlockSpec((BT, D), lambda i: (i, 0))],
        out_specs=pl.BlockSpec((BT, D), lambda i: (i, 0)),
        out_shape=jax.ShapeDtypeStruct((T, D), jnp.float32),
    )(s)


def _gather_gelu_body(g0_hbm, g1_hbm, g2_hbm, i0_hbm, i1_hbm, i2_hbm,
                      h_hbm,
                      i0a, i1a, i2a, r0a, r1a, r2a, sem_a,
                      i0b, i1b, i2b, r0b, r1b, r2b, sem_b):
    wid = lax.axis_index("s") * 2 + lax.axis_index("c")
    base = wid * NT
    bufs = ((i0a, i1a, i2a, r0a, r1a, r2a, sem_a),
            (i0b, i1b, i2b, r0b, r1b, r2b, sem_b))

    def fire(k, b):
        i0v, i1v, i2v, r0, r1, r2, sem = bufs[b]
        off = base + k * C
        pltpu.sync_copy(i0_hbm.at[pl.ds(off, C)], i0v)
        pltpu.sync_copy(i1_hbm.at[pl.ds(off, C)], i1v)
        pltpu.sync_copy(i2_hbm.at[pl.ds(off, C)], i2v)
        pltpu.async_copy(g0_hbm.at[i0v], r0, sem)
        pltpu.async_copy(g1_hbm.at[i1v], r1, sem)
        pltpu.async_copy(g2_hbm.at[i2v], r2, sem)

    def drain(b):
        _, _, _, r0, r1, r2, sem = bufs[b]
        pltpu.make_async_copy(g0_hbm.at[pl.ds(0, C)], r0, sem).wait()
        pltpu.make_async_copy(g1_hbm.at[pl.ds(0, C)], r1, sem).wait()
        pltpu.make_async_copy(g2_hbm.at[pl.ds(0, C)], r2, sem).wait()

    def compute_store(k, b):
        _, _, _, r0, r1, r2, _ = bufs[b]

        def row(i, carry2):
            for j in range(D // 16):
                sl = pl.ds(j * 16, 16)
                r0[i, sl] = r0[i, sl] + r1[i, sl] + r2[i, sl]
            return carry2

        lax.fori_loop(0, C, row, 0, unroll=4)
        pltpu.sync_copy(r0, h_hbm.at[pl.ds(base + k * C, C)])

    fire(0, 0)

    def pair(g, carry):
        k0 = g * 2
        fire(k0 + 1, 1)
        drain(0)
        compute_store(k0, 0)
        fire(k0 + 2, 0)
        drain(1)
        compute_store(k0 + 1, 1)
        return carry

    lax.fori_loop(0, (NCH - 1) // 2, pair, 0)
    drain(0)
    compute_store(NCH - 1, 0)


def _gather_gelu(g0, g1, g2, i0, i1, i2):
    ibuf = pltpu.VMEM((C,), jnp.int32)
    rbuf = pltpu.VMEM((C, D), jnp.float32)
    kern = pl.kernel(
        _gather_gelu_body,
        out_type=jax.ShapeDtypeStruct((T, D), jnp.float32),
        mesh=plsc.VectorSubcoreMesh(core_axis_name="c", subcore_axis_name="s"),
        scratch_types=[
            ibuf, ibuf, ibuf, rbuf, rbuf, rbuf, pltpu.SemaphoreType.DMA,
            ibuf, ibuf, ibuf, rbuf, rbuf, rbuf, pltpu.SemaphoreType.DMA,
        ],
    )
    return kern(g0, g1, g2, i0, i1, i2)


CH = 10880
NP = 15
EP = NP * 2 * CH
CB = 192
C2 = 8000
SCAN = T // 16
S = 4160
FB = 3840
MY = CH // 16
ZB = 136


def _scatter_body(h_hbm, i2_hbm, sums_hbm,
                  i2_v, cidx_st, cdst_st, cdst_b, rows, acc, sem):
    c = lax.axis_index("c")
    s = lax.axis_index("s")
    scan0 = s * SCAN
    my0 = s * MY

    def zero_rows(i, carry):
        for j in range(D // 16):
            rows[i, pl.ds(j * 16, 16)] = jnp.zeros((16,), jnp.float32)
        return carry

    def zero_my_slice():
        lax.fori_loop(0, ZB, zero_rows, 0)
        for k in range(MY // ZB):
            pltpu.sync_copy(rows.at[pl.ds(0, ZB)],
                            acc.at[pl.ds(my0 + k * ZB, ZB)])

    def do_batch(bi, carry2):
        for k in range(CB // 16):
            cdst_b[pl.ds(k * 16, 16)] = cdst_st[pl.ds(bi * CB + k * 16, 16)]
        pltpu.async_copy(h_hbm.at[cidx_st.at[pl.ds(bi * CB, CB)]],
                         rows, sem).wait()
        pltpu.sync_copy(rows, acc.at[cdst_b], add=True)
        return carry2

    zero_my_slice()
    plsc.subcore_barrier()

    def one_pass(p, carry):
        base = (p * 2 + c) * CH

        def chunkfn(k, cnt):
            off = scan0 + k * C2
            pltpu.sync_copy(i2_hbm.at[pl.ds(off, C2)], i2_v)

            def vec(j, cnt2):
                v = i2_v[pl.ds(j * 16, 16)]
                t = off + j * 16 + lax.iota(jnp.int32, 16)
                m = (v >= base) & (v < base + CH)
                plsc.store_compressed(cidx_st.at[pl.ds(cnt2, 16)], t, mask=m)
                plsc.store_compressed(cdst_st.at[pl.ds(cnt2, 16)], v - base,
                                      mask=m)
                cnt2 = cnt2 + jnp.sum(m.astype(jnp.int32))

                @pl.when(cnt2 >= FB)
                def flush():
                    lax.fori_loop(0, FB // CB, do_batch, 0)
                    cidx_st[pl.ds(0, 16)] = cidx_st[pl.ds(FB, 16)]
                    cdst_st[pl.ds(0, 16)] = cdst_st[pl.ds(FB, 16)]

                return jnp.where(cnt2 >= FB, cnt2 - FB, cnt2)

            return lax.fori_loop(0, C2 // 16, vec, cnt)

        cnt = lax.fori_loop(0, SCAN // C2, chunkfn, 0)

        spread = s * 16 + lax.iota(jnp.int32, 16)
        for kk in range(CB // 16):
            cidx_st[pl.ds(cnt + kk * 16, 16)] = spread
            cdst_st[pl.ds(cnt + kk * 16, 16)] = jnp.full((16,), CH, jnp.int32)
        nb = (cnt + CB - 1) // CB
        lax.fori_loop(0, nb, do_batch, 0)
        plsc.subcore_barrier()

        pltpu.sync_copy(acc.at[pl.ds(my0, MY)],
                        sums_hbm.at[pl.ds(base + my0, MY)])
        zero_my_slice()
        plsc.subcore_barrier()
        return carry

    lax.fori_loop(0, NP, one_pass, 0)


def _scatter(h, i2):
    kern = pl.kernel(
        _scatter_body,
        out_type=jax.ShapeDtypeStruct((EP, D), jnp.float32),
        mesh=plsc.VectorSubcoreMesh(core_axis_name="c", subcore_axis_name="s"),
        scratch_types=[
            pltpu.VMEM((C2,), jnp.int32),
            pltpu.VMEM((S,), jnp.int32),
            pltpu.VMEM((S,), jnp.int32),
            pltpu.VMEM((CB,), jnp.int32),
            pltpu.VMEM((CB, D), jnp.float32),
            pltpu.VMEM_SHARED((CH + 1, D), jnp.float32),
            pltpu.SemaphoreType.DMA,
        ],
        compiler_params=_SC_PARAMS,
    )
    return kern(h, i2)


def _finalize_body(s_ref, g_ref, bt_ref, o_ref):
    x = s_ref[...].astype(jnp.float32)
    mu = jnp.mean(x, axis=-1, keepdims=True)
    var = jnp.mean((x - mu) ** 2, axis=-1, keepdims=True)
    o_ref[...] = (x - mu) * lax.rsqrt(var + 1e-5) * g_ref[...] + bt_ref[...]


def _finalize(sums, gamma, beta):
    BE2 = 4000
    return pl.pallas_call(
        _finalize_body,
        grid=(E // BE2,),
        in_specs=[
            pl.BlockSpec((BE2, D), lambda i: (i, 0)),
            pl.BlockSpec((1, D), lambda i: (0, 0)),
            pl.BlockSpec((1, D), lambda i: (0, 0)),
        ],
        out_specs=pl.BlockSpec((BE2, D), lambda i: (i, 0)),
        out_shape=jax.ShapeDtypeStruct((E, D), jnp.float32),
    )(sums, gamma.reshape(1, D), beta.reshape(1, D))


def kernel(f_edge, triangle, W, b, gamma, beta):
    tri = triangle.astype(jnp.int32)
    i0 = tri[:, 0]
    i1 = tri[:, 1]
    i2 = tri[:, 2]
    Wt = W.T
    b2 = b.reshape(1, D)

    g0, g1, g2 = _matmul3(f_edge, Wt, b2)
    s = _gather_gelu(g0, g1, g2, i0, i1, i2)
    h = _gelu_dense(s)
    sums = _scatter(h, i2)
    return _finalize(sums, gamma, beta)


# speedup vs baseline: 1.1124x; 1.0561x over previous
import jax
import jax.numpy as jnp
from jax import lax
from jax.experimental import pallas as pl
from jax.experimental.pallas import tpu as pltpu
from jax.experimental.pallas import tpu_sc as plsc

E = 320000
D = 128
T = 640000

NW = 32
NT = T // NW
C = 160
NCH = NT // C

_SC_PARAMS = pltpu.CompilerParams(needs_layout_passes=False)


def _matmul3_body(f_ref, wt_ref, b_ref, o0, o1, o2):
    f = f_ref[...]
    o0[...] = jnp.dot(f, wt_ref[0:D, :], preferred_element_type=jnp.float32)
    o1[...] = jnp.dot(f, wt_ref[D:2 * D, :], preferred_element_type=jnp.float32)
    o2[...] = (jnp.dot(f, wt_ref[2 * D:3 * D, :],
                       preferred_element_type=jnp.float32) + b_ref[...])


def _matmul3(f_edge, Wt, b2):
    BE = 4000
    out = jax.ShapeDtypeStruct((E, D), jnp.float32)
    return pl.pallas_call(
        _matmul3_body,
        grid=(E // BE,),
        in_specs=[
            pl.BlockSpec((BE, D), lambda i: (i, 0)),
            pl.BlockSpec((3 * D, D), lambda i: (0, 0)),
            pl.BlockSpec((1, D), lambda i: (0, 0)),
        ],
        out_specs=[
            pl.BlockSpec((BE, D), lambda i: (i, 0)),
            pl.BlockSpec((BE, D), lambda i: (i, 0)),
            pl.BlockSpec((BE, D), lambda i: (i, 0)),
        ],
        out_shape=[out, out, out],
    )(f_edge, Wt, b2)


def _gelu_body(s_ref, o_ref):
    x = s_ref[...]
    o_ref[...] = 0.5 * x * (1.0 + lax.erf(x * 0.7071067811865476))


def _gelu_dense(s):
    BT = 4000
    return pl.pallas_call(
        _gelu_body,
        grid=(T // BT,),
        in_specs=[pl.BlockSpec((BT, D), lambda i: (i, 0))],
        out_specs=pl.BlockSpec((BT, D), lambda i: (i, 0)),
        out_shape=jax.ShapeDtypeStruct((T, D), jnp.float32),
    )(s)


def _gather_gelu_body(g0_hbm, g1_hbm, g2_hbm, i0_hbm, i1_hbm, i2_hbm,
                      h_hbm,
                      i0a, i1a, i2a, r0a, r1a, r2a, sem_a, isem_a,
                      i0b, i1b, i2b, r0b, r1b, r2b, sem_b, isem_b):
    wid = lax.axis_index("s") * 2 + lax.axis_index("c")
    base = wid * NT
    bufs = ((i0a, i1a, i2a, r0a, r1a, r2a, sem_a, isem_a),
            (i0b, i1b, i2b, r0b, r1b, r2b, sem_b, isem_b))

    def fire_idx(k, b):
        i0v, i1v, i2v, _, _, _, _, isem = bufs[b]
        off = base + jnp.minimum(k, NCH - 1) * C
        pltpu.async_copy(i0_hbm.at[pl.ds(off, C)], i0v, isem)
        pltpu.async_copy(i1_hbm.at[pl.ds(off, C)], i1v, isem)
        pltpu.async_copy(i2_hbm.at[pl.ds(off, C)], i2v, isem)

    def drain_idx(b):
        i0v, i1v, i2v, _, _, _, _, isem = bufs[b]
        pltpu.make_async_copy(i0_hbm.at[pl.ds(0, C)], i0v, isem).wait()
        pltpu.make_async_copy(i1_hbm.at[pl.ds(0, C)], i1v, isem).wait()
        pltpu.make_async_copy(i2_hbm.at[pl.ds(0, C)], i2v, isem).wait()

    def fire_g(b):
        i0v, i1v, i2v, r0, r1, r2, sem, _ = bufs[b]
        pltpu.async_copy(g0_hbm.at[i0v], r0, sem)
        pltpu.async_copy(g1_hbm.at[i1v], r1, sem)
        pltpu.async_copy(g2_hbm.at[i2v], r2, sem)

    def drain_g(b):
        _, _, _, r0, r1, r2, sem, _ = bufs[b]
        pltpu.make_async_copy(g0_hbm.at[pl.ds(0, C)], r0, sem).wait()
        pltpu.make_async_copy(g1_hbm.at[pl.ds(0, C)], r1, sem).wait()
        pltpu.make_async_copy(g2_hbm.at[pl.ds(0, C)], r2, sem).wait()

    def compute_store(k, b):
        _, _, _, r0, r1, r2, _, _ = bufs[b]

        def row(i, carry2):
            for j in range(D // 16):
                sl = pl.ds(j * 16, 16)
                r0[i, sl] = r0[i, sl] + r1[i, sl] + r2[i, sl]
            return carry2

        lax.fori_loop(0, C, row, 0, unroll=4)
        pltpu.sync_copy(r0, h_hbm.at[pl.ds(base + k * C, C)])

    fire_idx(0, 0)
    fire_idx(1, 1)
    drain_idx(0)
    fire_g(0)

    def pair(g, carry):
        k0 = g * 2
        drain_idx(1)
        fire_g(1)
        drain_g(0)
        fire_idx(k0 + 2, 0)
        compute_store(k0, 0)
        drain_idx(0)
        fire_g(0)
        drain_g(1)
        fire_idx(k0 + 3, 1)
        compute_store(k0 + 1, 1)
        return carry

    lax.fori_loop(0, (NCH - 1) // 2, pair, 0)
    drain_idx(1)
    drain_g(0)
    compute_store(NCH - 1, 0)


def _gather_gelu(g0, g1, g2, i0, i1, i2):
    ibuf = pltpu.VMEM((C,), jnp.int32)
    rbuf = pltpu.VMEM((C, D), jnp.float32)
    kern = pl.kernel(
        _gather_gelu_body,
        out_type=jax.ShapeDtypeStruct((T, D), jnp.float32),
        mesh=plsc.VectorSubcoreMesh(core_axis_name="c", subcore_axis_name="s"),
        scratch_types=[
            ibuf, ibuf, ibuf, rbuf, rbuf, rbuf,
            pltpu.SemaphoreType.DMA, pltpu.SemaphoreType.DMA,
            ibuf, ibuf, ibuf, rbuf, rbuf, rbuf,
            pltpu.SemaphoreType.DMA, pltpu.SemaphoreType.DMA,
        ],
    )
    return kern(g0, g1, g2, i0, i1, i2)


CH = 10880
NP = 15
EP = NP * 2 * CH
CB = 192
C2 = 8000
SCAN = T // 16
S = 4160
FB = 3840
MY = CH // 16
ZB = 136


def _scatter_body(h_hbm, i2_hbm, sums_hbm,
                  i2_v, cidx_st, cdst_st, cdst_b, rows, acc, sem):
    c = lax.axis_index("c")
    s = lax.axis_index("s")
    scan0 = s * SCAN
    my0 = s * MY

    def zero_rows(i, carry):
        for j in range(D // 16):
            rows[i, pl.ds(j * 16, 16)] = jnp.zeros((16,), jnp.float32)
        return carry

    def zero_my_slice():
        lax.fori_loop(0, ZB, zero_rows, 0)
        for k in range(MY // ZB):
            pltpu.sync_copy(rows.at[pl.ds(0, ZB)],
                            acc.at[pl.ds(my0 + k * ZB, ZB)])

    def do_batch(bi, carry2):
        for k in range(CB // 16):
            cdst_b[pl.ds(k * 16, 16)] = cdst_st[pl.ds(bi * CB + k * 16, 16)]
        pltpu.async_copy(h_hbm.at[cidx_st.at[pl.ds(bi * CB, CB)]],
                         rows, sem).wait()
        pltpu.sync_copy(rows, acc.at[cdst_b], add=True)
        return carry2

    zero_my_slice()
    plsc.subcore_barrier()

    def one_pass(p, carry):
        base = (p * 2 + c) * CH

        def chunkfn(k, cnt):
            off = scan0 + k * C2
            pltpu.sync_copy(i2_hbm.at[pl.ds(off, C2)], i2_v)

            def vec(j, cnt2):
                v = i2_v[pl.ds(j * 16, 16)]
                t = off + j * 16 + lax.iota(jnp.int32, 16)
                m = (v >= base) & (v < base + CH)
                plsc.store_compressed(cidx_st.at[pl.ds(cnt2, 16)], t, mask=m)
                plsc.store_compressed(cdst_st.at[pl.ds(cnt2, 16)], v - base,
                                      mask=m)
                cnt2 = cnt2 + jnp.sum(m.astype(jnp.int32))

                @pl.when(cnt2 >= FB)
                def flush():
                    lax.fori_loop(0, FB // CB, do_batch, 0)
                    cidx_st[pl.ds(0, 16)] = cidx_st[pl.ds(FB, 16)]
                    cdst_st[pl.ds(0, 16)] = cdst_st[pl.ds(FB, 16)]

                return jnp.where(cnt2 >= FB, cnt2 - FB, cnt2)

            return lax.fori_loop(0, C2 // 16, vec, cnt)

        cnt = lax.fori_loop(0, SCAN // C2, chunkfn, 0)

        spread = s * 16 + lax.iota(jnp.int32, 16)
        for kk in range(CB // 16):
            cidx_st[pl.ds(cnt + kk * 16, 16)] = spread
            cdst_st[pl.ds(cnt + kk * 16, 16)] = jnp.full((16,), CH, jnp.int32)
        nb = (cnt + CB - 1) // CB
        lax.fori_loop(0, nb, do_batch, 0)
        plsc.subcore_barrier()

        pltpu.sync_copy(acc.at[pl.ds(my0, MY)],
                        sums_hbm.at[pl.ds(base + my0, MY)])
        zero_my_slice()
        plsc.subcore_barrier()
        return carry

    lax.fori_loop(0, NP, one_pass, 0)


def _scatter(h, i2):
    kern = pl.kernel(
        _scatter_body,
        out_type=jax.ShapeDtypeStruct((EP, D), jnp.float32),
        mesh=plsc.VectorSubcoreMesh(core_axis_name="c", subcore_axis_name="s"),
        scratch_types=[
            pltpu.VMEM((C2,), jnp.int32),
            pltpu.VMEM((S,), jnp.int32),
            pltpu.VMEM((S,), jnp.int32),
            pltpu.VMEM((CB,), jnp.int32),
            pltpu.VMEM((CB, D), jnp.float32),
            pltpu.VMEM_SHARED((CH + 1, D), jnp.float32),
            pltpu.SemaphoreType.DMA,
        ],
        compiler_params=_SC_PARAMS,
    )
    return kern(h, i2)


def _finalize_body(s_ref, g_ref, bt_ref, o_ref):
    x = s_ref[...].astype(jnp.float32)
    mu = jnp.mean(x, axis=-1, keepdims=True)
    var = jnp.mean((x - mu) ** 2, axis=-1, keepdims=True)
    o_ref[...] = (x - mu) * lax.rsqrt(var + 1e-5) * g_ref[...] + bt_ref[...]


def _finalize(sums, gamma, beta):
    BE2 = 4000
    return pl.pallas_call(
        _finalize_body,
        grid=(E // BE2,),
        in_specs=[
            pl.BlockSpec((BE2, D), lambda i: (i, 0)),
            pl.BlockSpec((1, D), lambda i: (0, 0)),
            pl.BlockSpec((1, D), lambda i: (0, 0)),
        ],
        out_specs=pl.BlockSpec((BE2, D), lambda i: (i, 0)),
        out_shape=jax.ShapeDtypeStruct((E, D), jnp.float32),
    )(sums, gamma.reshape(1, D), beta.reshape(1, D))


def kernel(f_edge, triangle, W, b, gamma, beta):
    tri = triangle.astype(jnp.int32)
    i0 = tri[:, 0]
    i1 = tri[:, 1]
    i2 = tri[:, 2]
    Wt = W.T
    b2 = b.reshape(1, D)

    g0, g1, g2 = _matmul3(f_edge, Wt, b2)
    s = _gather_gelu(g0, g1, g2, i0, i1, i2)
    h = _gelu_dense(s)
    sums = _scatter(h, i2)
    return _finalize(sums, gamma, beta)
